# baseline (speedup 1.0000x reference)
import jax
import jax.numpy as jnp
from jax import lax
from jax.experimental import pallas as pl
from jax.experimental.pallas import tpu as pltpu
from jax.experimental.pallas import tpu_sc as plsc

D = 64
L = 16
CH = 128


def _build(B, S, n_time):
    rows_total = B * S
    info = plsc.get_sparse_core_info()
    NW = info.num_cores * info.num_subcores
    NC = info.num_cores
    RPW = rows_total // NW
    NCH = RPW // CH
    mesh = plsc.VectorSubcoreMesh(core_axis_name="c", subcore_axis_name="s")

    def body(ids_i, ids_t, w_word, w_pos, w_time, out_i, out_t, out_time,
             idx_a, idx_b, rows_a, rows_b, pos_v, tbuf, sem_a, sem_b, sem_p):
        wid = lax.axis_index("s") * NC + lax.axis_index("c")
        base = wid * RPW
        pos_base = lax.rem(base, S)

        cp_pos = pltpu.async_copy(w_pos.at[pl.ds(pos_base, RPW)], pos_v, sem_p)

        pltpu.sync_copy(ids_i.at[pl.ds(wid * NCH, NCH)], idx_a)
        pltpu.sync_copy(ids_t.at[pl.ds(wid * NCH, NCH)], idx_b)

        cps_a = [
            pltpu.async_copy(w_word.at[idx_a.at[j]],
                             rows_a.at[pl.ds(j * CH, CH)], sem_a)
            for j in range(NCH)
        ]
        cps_b = [
            pltpu.async_copy(w_word.at[idx_b.at[j]],
                             rows_b.at[pl.ds(j * CH, CH)], sem_b)
            for j in range(NCH)
        ]
        cp_pos.wait()

        def add_pos(rows):
            def row_fn(r, carry):
                for k in range(D // L):
                    sl = pl.ds(k * L, L)
                    rows[r, sl] = rows[r, sl] + pos_v[r, sl]
                return carry
            lax.fori_loop(0, RPW, row_fn, 0, unroll=2)

        for cp in cps_a:
            cp.wait()
        add_pos(rows_a)
        pltpu.sync_copy(rows_a, out_i.at[pl.ds(base, RPW)])

        for cp in cps_b:
            cp.wait()
        add_pos(rows_b)
        pltpu.sync_copy(rows_b, out_t.at[pl.ds(base, RPW)])

        @pl.when(wid == 0)
        def _():
            pltpu.sync_copy(w_time, tbuf)
            pltpu.sync_copy(tbuf, out_time.at[0])
            pltpu.sync_copy(tbuf, out_time.at[1])

    return pl.kernel(
        body,
        out_type=(
            jax.ShapeDtypeStruct((rows_total, D), jnp.float32),
            jax.ShapeDtypeStruct((rows_total, D), jnp.float32),
            jax.ShapeDtypeStruct((2, n_time, D), jnp.float32),
        ),
        mesh=mesh,
        compiler_params=pltpu.CompilerParams(use_tc_tiling_on_sc=False),
        scratch_types=[
            pltpu.VMEM((NCH, CH), jnp.int32),
            pltpu.VMEM((NCH, CH), jnp.int32),
            pltpu.VMEM((RPW, D), jnp.float32),
            pltpu.VMEM((RPW, D), jnp.float32),
            pltpu.VMEM((RPW, D), jnp.float32),
            pltpu.VMEM((n_time, D), jnp.float32),
            pltpu.SemaphoreType.DMA,
            pltpu.SemaphoreType.DMA,
            pltpu.SemaphoreType.DMA,
        ],
    )


def kernel(input_ids, target_ids, W_word, W_pos, W_time):
    B, S = input_ids.shape
    n_time = W_time.shape[0]
    rows_total = B * S
    ids_i = input_ids.astype(jnp.int32).reshape(rows_total // CH, CH)
    ids_t = target_ids.astype(jnp.int32).reshape(rows_total // CH, CH)
    k = _build(B, S, n_time)
    out_i, out_t, out_time = k(ids_i, ids_t, W_word, W_pos, W_time)
    return (out_i.reshape(B, S, D), out_t.reshape(B, S, D),
            out_time.reshape(1, 2 * n_time, D))
